# baseline (speedup 1.0000x reference)
import jax
import jax.numpy as jnp
from jax import lax
from jax.experimental import pallas as pl
from jax.experimental.pallas import tpu as pltpu
from jax.experimental.pallas import tpu_sc as plsc

BATCH = 128
VOCAB = 100000
K = 8


ROWS_PER_BLOCK = 8
NUM_BLOCKS = BATCH // ROWS_PER_BLOCK


W = 128
NUM_SUBS = -(-VOCAB // W)
N_CHAINS = 4


def _sub(x, u):
    lo = u * W
    if lo + W <= VOCAB:
        return x[:, lo:lo + W]
    pad = jnp.full((x.shape[0], lo + W - VOCAB), -jnp.inf, jnp.float32)
    return jnp.concatenate([x[:, lo:VOCAB], pad], axis=1)


def _lexgt(v1, i1, v2, i2):
    return (v1 > v2) | ((v1 == v2) & (i1 < i2))


def _merge2(a, b):
    av1, ai1, av2, ai2 = a
    bv1, bi1, bv2, bi2 = b
    t = _lexgt(bv1, bi1, av1, ai1)
    w1v = jnp.where(t, bv1, av1)
    w1i = jnp.where(t, bi1, ai1)
    l1v = jnp.where(t, av1, bv1)
    l1i = jnp.where(t, ai1, bi1)
    c = _lexgt(bv2, bi2, av2, ai2)
    w2v = jnp.where(c, bv2, av2)
    w2i = jnp.where(c, bi2, ai2)
    d = _lexgt(l1v, l1i, w2v, w2i)
    s2v = jnp.where(d, l1v, w2v)
    s2i = jnp.where(d, l1i, w2i)
    return (w1v, w1i, s2v, s2i)


def _topk_mask_body(x_ref, masked_ref, vals_ref, idx_ref, dirty_ref):
    x = x_ref[...]
    rows = x.shape[0]
    neg_inf = jnp.float32(-jnp.inf)
    big = jnp.int32(1 << 30)
    iota_w = lax.broadcasted_iota(jnp.int32, (rows, W), 1)

    SW = 8 * W
    n_slabs = -(-VOCAB // SW)

    def _slab(xv, s):
        lo = s * SW
        if lo + SW <= VOCAB:
            return xv[:, lo:lo + SW]
        pad = jnp.full((rows, lo + SW - VOCAB), neg_inf, jnp.float32)
        return jnp.concatenate([xv[:, lo:VOCAB], pad], axis=1)

    curv = jnp.full((rows, SW), neg_inf, jnp.float32)
    curs = jnp.zeros((rows, SW), jnp.int32)
    nxtv = jnp.full((rows, SW), neg_inf, jnp.float32)
    nxts = jnp.zeros((rows, SW), jnp.int32)
    for s in range(n_slabs):
        slab = _slab(x, s)
        s32 = jnp.int32(s)
        b1 = slab > curv
        new2v = jnp.where(b1, curv, slab)
        new2s = jnp.where(b1, curs, s32)
        curs = jnp.where(b1, s32, curs)
        curv = jnp.maximum(curv, slab)
        b2 = slab > nxtv
        nxtv = jnp.where(b2, new2v, nxtv)
        nxts = jnp.where(b2, new2s, nxts)

    tiles = []
    for t in range(8):
        sl = slice(t * W, (t + 1) * W)
        tiles.append((curv[:, sl], curs[:, sl] * 8 + t,
                      nxtv[:, sl], nxts[:, sl] * 8 + t))
    while len(tiles) > 1:
        tiles = [_merge2(tiles[i], tiles[i + 1])
                 for i in range(0, len(tiles), 2)]
    curv, curs, nxtv, nxts = tiles[0]

    def _round(cv, cs, nv, ns, hv):
        m = jnp.max(cv, axis=1, keepdims=True)
        jcand = cs * W + iota_w
        cand = jnp.where(cv == m, jcand, big)
        j = jnp.min(cand, axis=1, keepdims=True)
        onehot = (cv == m) & (jcand == j)
        ex = onehot & (hv == 0)
        cv = jnp.where(onehot, nv, cv)
        cs = jnp.where(onehot, ns, cs)
        hv = jnp.where(onehot, 0, hv)
        return m, j, onehot, ex, cv, cs, hv

    have = jnp.ones((rows, W), jnp.int32)
    vals_cols = []
    idx_cols = []
    exacc = jnp.zeros((rows, W), jnp.int32)
    for _ in range(K):
        m, j, _, ex, curv, curs, have = _round(curv, curs, nxtv, nxts, have)
        vals_cols.append(m)
        idx_cols.append(j)
        exacc = exacc | ex.astype(jnp.int32)
    vals = jnp.concatenate(vals_cols, axis=1)
    idxs = jnp.concatenate(idx_cols, axis=1)

    thresh = vals[:, K - 1:K]
    masked_ref[...] = jnp.where(x >= thresh, x, jnp.finfo(jnp.float32).min)
    vals_ref[...] = vals
    idx_ref[...] = idxs
    dirty_ref[...] = jnp.max(exacc, axis=1, keepdims=True)


def _topk_mask(logits):
    return pl.pallas_call(
        _topk_mask_body,
        grid=(NUM_BLOCKS,),
        in_specs=[pl.BlockSpec((ROWS_PER_BLOCK, VOCAB), lambda i: (i, 0))],
        out_specs=[
            pl.BlockSpec((ROWS_PER_BLOCK, VOCAB), lambda i: (i, 0)),
            pl.BlockSpec((ROWS_PER_BLOCK, K), lambda i: (i, 0)),
            pl.BlockSpec((ROWS_PER_BLOCK, K), lambda i: (i, 0)),
            pl.BlockSpec((ROWS_PER_BLOCK, 1), lambda i: (i, 0)),
        ],
        out_shape=[
            jax.ShapeDtypeStruct((BATCH, VOCAB), jnp.float32),
            jax.ShapeDtypeStruct((BATCH, K), jnp.float32),
            jax.ShapeDtypeStruct((BATCH, K), jnp.int32),
            jax.ShapeDtypeStruct((BATCH, 1), jnp.int32),
        ],
    )(logits)


N_GROUPS = BATCH // 8


def _fix_body(gflags_ref, logits_ref, masked_in_ref, vals_in_ref, idx_in_ref,
              masked_out_ref, vals_out_ref, idx_out_ref, xscr, sem):
    vals_out_ref[...] = vals_in_ref[...]
    idx_out_ref[...] = idx_in_ref[...]
    del masked_in_ref
    neg_inf = jnp.float32(-jnp.inf)
    big = jnp.int32(1 << 30)
    for g in range(N_GROUPS):
        @pl.when(gflags_ref[g] != 0)
        def _():
            cin = pltpu.make_async_copy(
                logits_ref.at[pl.ds(g * 8, 8), :], xscr, sem)
            cin.start()
            cin.wait()
            x = xscr[...]
            iota = lax.broadcasted_iota(jnp.int32, x.shape, 1)
            xc = x
            vcols, icols = [], []
            for _ in range(K):
                m = jnp.max(xc, axis=1, keepdims=True)
                j = jnp.min(jnp.where(xc == m, iota, big), axis=1,
                            keepdims=True)
                vcols.append(m)
                icols.append(j)
                xc = jnp.where(iota == j, neg_inf, xc)
            vals8 = jnp.concatenate(vcols, axis=1)
            idx8 = jnp.concatenate(icols, axis=1)
            vals_out_ref[pl.ds(g * 8, 8), :] = vals8
            idx_out_ref[pl.ds(g * 8, 8), :] = idx8
            xscr[...] = jnp.where(x >= vals8[:, K - 1:K], x,
                                  jnp.finfo(jnp.float32).min)
            cout = pltpu.make_async_copy(
                xscr, masked_out_ref.at[pl.ds(g * 8, 8), :], sem)
            cout.start()
            cout.wait()


def _fixup(dirty, logits, masked, vals, idxs):
    gflags = jnp.max(dirty.reshape(N_GROUPS, 8), axis=1)
    return pl.pallas_call(
        _fix_body,
        in_specs=[
            pl.BlockSpec(memory_space=pltpu.SMEM),
            pl.BlockSpec(memory_space=pltpu.MemorySpace.HBM),
            pl.BlockSpec(memory_space=pltpu.MemorySpace.HBM),
            pl.BlockSpec((BATCH, K), lambda: (0, 0)),
            pl.BlockSpec((BATCH, K), lambda: (0, 0)),
        ],
        out_specs=[
            pl.BlockSpec(memory_space=pltpu.MemorySpace.HBM),
            pl.BlockSpec((BATCH, K), lambda: (0, 0)),
            pl.BlockSpec((BATCH, K), lambda: (0, 0)),
        ],
        out_shape=[
            jax.ShapeDtypeStruct((BATCH, VOCAB), jnp.float32),
            jax.ShapeDtypeStruct((BATCH, K), jnp.float32),
            jax.ShapeDtypeStruct((BATCH, K), jnp.int32),
        ],
        input_output_aliases={2: 0},
        scratch_shapes=[
            pltpu.VMEM((8, VOCAB), jnp.float32),
            pltpu.SemaphoreType.DMA,
        ],
    )(gflags, logits, masked, vals, idxs)


ADJ_BLOCKS = 20
ADJ_BLOCK_ROWS = VOCAB // ADJ_BLOCKS


def _adj_body(tok_ref, idx_ref, adj_ref, out_ref):
    i = pl.program_id(0)
    out_ref[...] = adj_ref[...]
    base = i * ADJ_BLOCK_ROWS

    def write_one(t_i, carry):
        r = tok_ref[t_i] - base

        @pl.when((r >= 0) & (r < ADJ_BLOCK_ROWS))
        def _():
            out_ref[pl.ds(r, 1), :] = idx_ref[pl.ds(t_i, 1), :]

        return carry

    lax.fori_loop(0, BATCH, write_one, 0)


def _adj_update(adjacency, tokens, idx):
    return pl.pallas_call(
        _adj_body,
        grid=(ADJ_BLOCKS,),
        in_specs=[
            pl.BlockSpec(memory_space=pltpu.SMEM),
            pl.BlockSpec((BATCH, K), lambda i: (0, 0)),
            pl.BlockSpec((ADJ_BLOCK_ROWS, K), lambda i: (i, 0)),
        ],
        out_specs=pl.BlockSpec((ADJ_BLOCK_ROWS, K), lambda i: (i, 0)),
        out_shape=jax.ShapeDtypeStruct((VOCAB, K), jnp.int32),
    )(tokens, idx, adjacency)


def kernel(logits, tokens, adjacency, k):
    masked_logits, vals, idx, dirty = _topk_mask(logits)
    masked_logits, vals, idx = _fixup(dirty, logits, masked_logits, vals, idx)
    k_static = adjacency.shape[1]
    idx = (idx + (k - k_static)).astype(jnp.int32)
    new_adjacency = _adj_update(adjacency, tokens, idx)
    return masked_logits, vals, idx, new_adjacency


# speedup vs baseline: 1.8546x; 1.8546x over previous
import jax
import jax.numpy as jnp
from jax import lax
from jax.experimental import pallas as pl
from jax.experimental.pallas import tpu as pltpu

BATCH = 128
VOCAB = 100000
K = 8


ROWS_PER_BLOCK = 8
NUM_BLOCKS = BATCH // ROWS_PER_BLOCK

W = 1024
NUM_SLABS = -(-VOCAB // W)


def _slab(x, s):
    lo = s * W
    if lo + W <= VOCAB:
        return x[:, lo:lo + W]
    pad = jnp.full((x.shape[0], lo + W - VOCAB), -jnp.inf, jnp.float32)
    return jnp.concatenate([x[:, lo:VOCAB], pad], axis=1)


def _topk_mask_body(x_ref, masked_ref, vals_ref, idx_ref):
    x = x_ref[...]
    rows = x.shape[0]
    neg_inf = jnp.float32(-jnp.inf)
    big = jnp.int32(1 << 30)
    iota_w = lax.broadcasted_iota(jnp.int32, (rows, W), 1)

    curv = jnp.full((rows, W), neg_inf, jnp.float32)
    curs = jnp.zeros((rows, W), jnp.int32)
    nxtv = jnp.full((rows, W), neg_inf, jnp.float32)
    nxts = jnp.zeros((rows, W), jnp.int32)
    for s in range(NUM_SLABS):
        slab = _slab(x, s)
        s32 = jnp.int32(s)
        b1 = slab > curv
        new2v = jnp.where(b1, curv, slab)
        new2s = jnp.where(b1, curs, s32)
        curs = jnp.where(b1, s32, curs)
        curv = jnp.maximum(curv, slab)
        b2 = slab > nxtv
        nxtv = jnp.where(b2, new2v, nxtv)
        nxts = jnp.where(b2, new2s, nxts)

    have = jnp.ones((rows, W), jnp.int32)
    vals_cols = []
    idx_cols = []
    for _ in range(K):
        m = jnp.max(curv, axis=1, keepdims=True)
        jcand = curs * W + iota_w
        cand = jnp.where(curv == m, jcand, big)
        j = jnp.min(cand, axis=1, keepdims=True)
        vals_cols.append(m)
        idx_cols.append(j)
        onehot = (curv == m) & (jcand == j)
        ex = onehot & (have == 0)
        curv = jnp.where(onehot, nxtv, curv)
        curs = jnp.where(onehot, nxts, curs)
        have = jnp.where(onehot, 0, have)

        def _fallback(args):
            curv, curs, ex, m, j = args
            cstar = jnp.min(jnp.where(ex, iota_w, big), axis=1, keepdims=True)
            nv = jnp.full((rows, 1), neg_inf, jnp.float32)
            ns = jnp.zeros((rows, 1), jnp.int32)
            for s in range(NUM_SLABS):
                slab = _slab(x, s)
                eidx = jnp.int32(s * W) + iota_w
                lexless = (slab < m) | ((slab == m) & (eidx > j))
                valid = (iota_w == cstar) & lexless
                v = jnp.max(jnp.where(valid, slab, neg_inf), axis=1,
                            keepdims=True)
                b = v > nv
                ns = jnp.where(b, jnp.int32(s), ns)
                nv = jnp.maximum(nv, v)
            return jnp.where(ex, nv, curv), jnp.where(ex, ns, curs)

        curv, curs = lax.cond(jnp.any(ex), _fallback,
                              lambda args: (args[0], args[1]),
                              (curv, curs, ex, m, j))

    thresh = vals_cols[-1]
    masked_ref[...] = jnp.where(x >= thresh, x, jnp.finfo(jnp.float32).min)
    vals_ref[...] = jnp.concatenate(vals_cols, axis=1)
    idx_ref[...] = jnp.concatenate(idx_cols, axis=1)


def _topk_mask(logits):
    return pl.pallas_call(
        _topk_mask_body,
        grid=(NUM_BLOCKS,),
        in_specs=[pl.BlockSpec((ROWS_PER_BLOCK, VOCAB), lambda i: (i, 0))],
        out_specs=[
            pl.BlockSpec((ROWS_PER_BLOCK, VOCAB), lambda i: (i, 0)),
            pl.BlockSpec((ROWS_PER_BLOCK, K), lambda i: (i, 0)),
            pl.BlockSpec((ROWS_PER_BLOCK, K), lambda i: (i, 0)),
        ],
        out_shape=[
            jax.ShapeDtypeStruct((BATCH, VOCAB), jnp.float32),
            jax.ShapeDtypeStruct((BATCH, K), jnp.float32),
            jax.ShapeDtypeStruct((BATCH, K), jnp.int32),
        ],
    )(logits)


ADJ_BLOCKS = 20
ADJ_BLOCK_ROWS = VOCAB // ADJ_BLOCKS


def _adj_body(tok_ref, idx_ref, adj_ref, out_ref):
    i = pl.program_id(0)
    out_ref[...] = adj_ref[...]
    base = i * ADJ_BLOCK_ROWS

    def write_one(t_i, carry):
        r = tok_ref[t_i] - base

        @pl.when((r >= 0) & (r < ADJ_BLOCK_ROWS))
        def _():
            out_ref[pl.ds(r, 1), :] = idx_ref[pl.ds(t_i, 1), :]

        return carry

    lax.fori_loop(0, BATCH, write_one, 0)


def _adj_update(adjacency, tokens, idx):
    return pl.pallas_call(
        _adj_body,
        grid=(ADJ_BLOCKS,),
        in_specs=[
            pl.BlockSpec(memory_space=pltpu.SMEM),
            pl.BlockSpec((BATCH, K), lambda i: (0, 0)),
            pl.BlockSpec((ADJ_BLOCK_ROWS, K), lambda i: (i, 0)),
        ],
        out_specs=pl.BlockSpec((ADJ_BLOCK_ROWS, K), lambda i: (i, 0)),
        out_shape=jax.ShapeDtypeStruct((VOCAB, K), jnp.int32),
    )(tokens, idx, adjacency)


def kernel(logits, tokens, adjacency, k):
    masked_logits, vals, idx = _topk_mask(logits)
    k_static = adjacency.shape[1]
    idx = (idx + (k - k_static)).astype(jnp.int32)
    new_adjacency = _adj_update(adjacency, tokens, idx)
    return masked_logits, vals, idx, new_adjacency
